# Initial kernel scaffold; baseline (speedup 1.0000x reference)
import jax
import jax.numpy as jnp
from jax.experimental import pallas as pl


def kernel(tokens):
    raise NotImplementedError("write your pallas kernel here")


# speedup vs baseline: 1.4545x; 1.4545x over previous
import jax
import jax.numpy as jnp
from jax.experimental import pallas as pl
from jax.experimental.pallas import tpu as pltpu

FREEZE_RATIO = 0.9
ROWS_PER_BLOCK = 512


def _freeze_body(target_smem, x_ref, out_ref, count_ref):
    i = pl.program_id(0)
    target = target_smem[0]
    row0 = i * ROWS_PER_BLOCK
    row_end = row0 + ROWS_PER_BLOCK

    @pl.when(i == 0)
    def _():
        count_ref[0] = target

    @pl.when(row_end <= target)
    def _():
        out_ref[...] = jnp.zeros_like(out_ref)

    @pl.when(row0 >= target)
    def _():
        out_ref[...] = x_ref[...]

    @pl.when(jnp.logical_and(row0 < target, row_end > target))
    def _():
        rows = row0 + jax.lax.broadcasted_iota(
            jnp.int32, out_ref.shape, 0
        )
        out_ref[...] = jnp.where(rows < target, 0.0, x_ref[...])


def kernel(tokens):
    batch_size, seq_len, d_model = tokens.shape
    total = batch_size * seq_len
    target = int(total * FREEZE_RATIO)
    assert total % ROWS_PER_BLOCK == 0
    num_blocks = total // ROWS_PER_BLOCK
    first_live = target // ROWS_PER_BLOCK

    x = tokens.reshape(total, d_model)

    frozen_flat, count = pl.pallas_call(
        _freeze_body,
        grid=(num_blocks,),
        in_specs=[
            pl.BlockSpec(memory_space=pltpu.SMEM),
            pl.BlockSpec(
                (ROWS_PER_BLOCK, d_model),
                lambda i: (jnp.maximum(i, first_live), 0),
            ),
        ],
        out_specs=[
            pl.BlockSpec((ROWS_PER_BLOCK, d_model), lambda i: (i, 0)),
            pl.BlockSpec(memory_space=pltpu.SMEM),
        ],
        out_shape=[
            jax.ShapeDtypeStruct((total, d_model), tokens.dtype),
            jax.ShapeDtypeStruct((1,), jnp.int32),
        ],
    )(jnp.full((1,), target, dtype=jnp.int32), x)

    frozen_tokens = frozen_flat.reshape(batch_size, seq_len, d_model)
    return (frozen_tokens, count[0])


# speedup vs baseline: 1.6305x; 1.1210x over previous
import jax
import jax.numpy as jnp
from jax.experimental import pallas as pl
from jax.experimental.pallas import tpu as pltpu

FREEZE_RATIO = 0.9
ROWS_PER_BLOCK = 1024


def _freeze_body(target_smem, x_ref, out_ref, count_ref):
    i = pl.program_id(0)
    target = target_smem[0]
    row0 = i * ROWS_PER_BLOCK
    row_end = row0 + ROWS_PER_BLOCK

    @pl.when(i == 0)
    def _():
        count_ref[0] = target

    @pl.when(row_end <= target)
    def _():
        out_ref[...] = jnp.zeros_like(out_ref)

    @pl.when(row0 >= target)
    def _():
        out_ref[...] = x_ref[...]

    @pl.when(jnp.logical_and(row0 < target, row_end > target))
    def _():
        rows = row0 + jax.lax.broadcasted_iota(
            jnp.int32, out_ref.shape, 0
        )
        out_ref[...] = jnp.where(rows < target, 0.0, x_ref[...])


def kernel(tokens):
    batch_size, seq_len, d_model = tokens.shape
    total = batch_size * seq_len
    target = int(total * FREEZE_RATIO)
    assert total % ROWS_PER_BLOCK == 0
    num_blocks = total // ROWS_PER_BLOCK
    first_live = target // ROWS_PER_BLOCK

    x = tokens.reshape(total, d_model)

    frozen_flat, count = pl.pallas_call(
        _freeze_body,
        grid=(num_blocks,),
        in_specs=[
            pl.BlockSpec(memory_space=pltpu.SMEM),
            pl.BlockSpec(
                (ROWS_PER_BLOCK, d_model),
                lambda i: (jnp.maximum(i, first_live), 0),
            ),
        ],
        out_specs=[
            pl.BlockSpec((ROWS_PER_BLOCK, d_model), lambda i: (i, 0)),
            pl.BlockSpec(memory_space=pltpu.SMEM),
        ],
        out_shape=[
            jax.ShapeDtypeStruct((total, d_model), tokens.dtype),
            jax.ShapeDtypeStruct((1,), jnp.int32),
        ],
    )(jnp.full((1,), target, dtype=jnp.int32), x)

    frozen_tokens = frozen_flat.reshape(batch_size, seq_len, d_model)
    return (frozen_tokens, count[0])
